# baseline (speedup 1.0000x reference)
import jax
import jax.numpy as jnp
from jax.experimental import pallas as pl

N = 8
C_IN = 384
H = 56
W = 56
OUT = 384
N_HASHES = 4
CHUNK = 14
GROUPS = 8
KSIZE = 56
GP = OUT // GROUPS
F_GW = 0.1
F_GV1 = 0.1
F_GV2 = 1.0

B = N * W
COLS = B * H
QKV = OUT * 3 // 2
WM = OUT // 2
NB = 4
NEG = -1e30


def _qkv_kernel(w_ref, x_ref, o_ref, sum_ref, sq_ref):
    q = jnp.dot(w_ref[...], x_ref[...], preferred_element_type=jnp.float32)
    o_ref[...] = q
    s = jnp.sum(q, axis=1, keepdims=True)
    s2 = jnp.sum(q * q, axis=1, keepdims=True)

    @pl.when(pl.program_id(0) == 0)
    def _init():
        sum_ref[...] = s
        sq_ref[...] = s2

    @pl.when(pl.program_id(0) != 0)
    def _acc():
        sum_ref[...] += s
        sq_ref[...] += s2


def _qkv_bn(x, conv_w):
    xq = jnp.transpose(x, (1, 0, 3, 2)).reshape(C_IN, COLS)
    bw = 3584 if COLS % 3584 == 0 else COLS
    grid = COLS // bw
    qkv_all, ssum, ssq = pl.pallas_call(
        _qkv_kernel,
        grid=(grid,),
        in_specs=[
            pl.BlockSpec((QKV, C_IN), lambda i: (0, 0)),
            pl.BlockSpec((C_IN, bw), lambda i: (0, i)),
        ],
        out_specs=[
            pl.BlockSpec((QKV, bw), lambda i: (0, i)),
            pl.BlockSpec((QKV, 1), lambda i: (0, 0)),
            pl.BlockSpec((QKV, 1), lambda i: (0, 0)),
        ],
        out_shape=[
            jax.ShapeDtypeStruct((QKV, COLS), jnp.float32),
            jax.ShapeDtypeStruct((QKV, 1), jnp.float32),
            jax.ShapeDtypeStruct((QKV, 1), jnp.float32),
        ],
    )(conv_w, xq)
    mean = ssum[:, 0] / COLS
    var = ssq[:, 0] / COLS - mean * mean
    return qkv_all, mean, var


BBS = 4


def _sort_kernel(qkv_ref, sm_ref, sh_ref, rot_ref, sw_ref, sv_ref, pos_ref):
    rows = jax.lax.broadcasted_iota(jnp.int32, (H, H), 0)
    cols = jax.lax.broadcasted_iota(jnp.int32, (H, H), 1)
    lmat = (rows >= cols).astype(jnp.float32)
    su4 = (jax.lax.broadcasted_iota(jnp.int32, (NB, NB), 0) <
           jax.lax.broadcasted_iota(jnp.int32, (NB, NB), 1)).astype(jnp.float32)
    lane4 = jax.lax.broadcasted_iota(jnp.int32, (H, NB), 1)
    lane56 = jax.lax.broadcasted_iota(jnp.int32, (H, H), 1)
    sm = sm_ref[...]
    sh = sh_ref[...]
    rot = rot_ref[...]
    for lb in range(BBS):
        qn = qkv_ref[lb] * sm + sh
        wm = qn[:, :WM]
        rotated = jnp.dot(wm, rot,
                          preferred_element_type=jnp.float32)
        pos_cols = []
        for h in range(N_HASHES):
            l0 = rotated[:, 2 * h:2 * h + 1]
            l1 = rotated[:, 2 * h + 1:2 * h + 2]
            best = l0
            bi = jnp.zeros((H, 1), jnp.int32)
            for j, v in ((1, l1), (2, -l0), (3, -l1)):
                upd = v > best
                bi = jnp.where(upd, j, bi)
                best = jnp.maximum(best, v)
            onehot = (bi == lane4).astype(jnp.float32)
            csum = jnp.dot(lmat, onehot,
                           preferred_element_type=jnp.float32)
            totals = csum[H - 1:H, :]
            offs = jnp.dot(totals, su4,
                           preferred_element_type=jnp.float32)
            posf = jnp.sum(onehot * (offs + csum), axis=1,
                           keepdims=True) - 1.0
            pos_i = posf.astype(jnp.int32)
            pos_cols.append(pos_i)
            pt = (pos_i == lane56).astype(jnp.float32)
            sorted_h = jax.lax.dot_general(
                pt, qn, (((0,), (0,)), ((), ())),
                preferred_element_type=jnp.float32)
            sw_ref[lb, h] = sorted_h[:, :WM]
            sv_ref[lb, h] = sorted_h[:, WM:]
        pos_ref[lb] = jnp.concatenate(pos_cols, axis=1)


def _sort_gather(qkv2, scale_map, shift_map, rotf):
    return pl.pallas_call(
        _sort_kernel,
        grid=(B // BBS,),
        in_specs=[
            pl.BlockSpec((BBS, H, QKV), lambda i: (i, 0, 0)),
            pl.BlockSpec((H, QKV), lambda i: (0, 0)),
            pl.BlockSpec((H, QKV), lambda i: (0, 0)),
            pl.BlockSpec((WM, 2 * N_HASHES), lambda i: (0, 0)),
        ],
        out_specs=[
            pl.BlockSpec((BBS, N_HASHES, H, WM), lambda i: (i, 0, 0, 0)),
            pl.BlockSpec((BBS, N_HASHES, H, OUT), lambda i: (i, 0, 0, 0)),
            pl.BlockSpec((BBS, H, N_HASHES), lambda i: (i, 0, 0)),
        ],
        out_shape=[
            jax.ShapeDtypeStruct((B, N_HASHES, H, WM), jnp.float32),
            jax.ShapeDtypeStruct((B, N_HASHES, H, OUT), jnp.float32),
            jax.ShapeDtypeStruct((B, H, N_HASHES), jnp.int32),
        ],
    )(qkv2, scale_map, shift_map, rotf)


def _attn_kernel(ws_ref, vs_ref, we_ref, ve_ref, m7_ref, gb_ref, gbt_ref,
                 out_ref, bsn_ref):
    wes = we_ref[...] * F_GW
    ves = ve_ref[...] * F_GV1
    ci = jax.lax.broadcasted_iota(jnp.int32, (H, H), 0) // CHUNK
    cj = jax.lax.broadcasted_iota(jnp.int32, (H, H), 1) // CHUNK
    allowed = cj != ((ci + 2) & 3)

    qstack = ws_ref[0].reshape(8 * N_HASHES * H, GP // 2)
    nrm = jnp.sqrt(jnp.sum(qstack * qstack, axis=1, keepdims=True))
    qn_stack = qstack / jnp.maximum(nrm, 5e-5)

    def _blk(arr, h, au):
        base = h * 8 * H + au * H
        return arr[base:base + H, :]

    sa_list = []
    for g in range(GROUPS):
        h, a = g // 2, g % 2
        for u in range(N_HASHES):
            au = a * 4 + u
            q = _blk(qstack, h, au)
            rhs = jnp.concatenate([_blk(qn_stack, h, au), wes], axis=0)
            sa_list.append(jax.lax.dot_general(
                q, rhs, (((1,), (1,)), ((), ())),
                preferred_element_type=jnp.float32))
    sa = jnp.concatenate(sa_list, axis=0).reshape(GROUPS, N_HASHES, H, 2 * H)
    ssq = jnp.sum(sa * sa, axis=0)
    den = jnp.maximum(jnp.sqrt(ssq[:, :, :H] + ssq[:, :, H:]), 5e-5)
    ss = (sa[..., :H] + sa[..., H:]) / den[None]
    ssm = jnp.where(allowed[None, None], ss, NEG)
    m = jnp.max(ssm, axis=-1, keepdims=True)
    e = jnp.exp(ssm - m)
    se = jnp.sum(e, axis=-1, keepdims=True)
    bs_all = m + jnp.log(se)
    p_all = e / se

    r96 = []
    for g in range(GROUPS):
        h, a = g // 2, g % 2
        for u in range(N_HASHES):
            au = a * 4 + u
            vcat = jnp.concatenate(
                [vs_ref[0, h][au * H:(au + 1) * H, :], ves], axis=1)
            r96.append(jnp.dot(p_all[g, u], vcat,
                               preferred_element_type=jnp.float32))
    rbig = jnp.concatenate(r96, axis=0)

    m7 = m7_ref[...]
    ssq = jnp.dot(m7, rbig * rbig,
                  preferred_element_type=jnp.float32)
    norms = jnp.zeros((1, 224), jnp.float32)
    for b in range(7):
        norms += jnp.dot(ssq[b:b + 1, :], gb_ref[b],
                         preferred_element_type=jnp.float32)
    recip = 1.0 / jnp.maximum(jnp.sqrt(norms), 5e-5)
    drows = []
    for b in range(7):
        drows.append(jnp.dot(recip, gbt_ref[b],
                             preferred_element_type=jnp.float32))
    drows.append(jnp.zeros((1, 2 * GP), jnp.float32))
    recip_block = jnp.concatenate(drows, axis=0)
    recip_full = jax.lax.dot_general(
        m7, recip_block, (((0,), (0,)), ((), ())),
        preferred_element_type=jnp.float32)
    out_ref[0] = rbig * recip_full

    sq = jnp.sum(bs_all * bs_all, axis=0)
    nb = jnp.maximum(jnp.sqrt(sq), 5e-5)
    acc = jnp.sum(bs_all, axis=0) / nb
    bsn_ref[0] = jnp.concatenate([acc[u] for u in range(N_HASHES)], axis=1)


def _attention(ws_r, vs_r, we_full, ve_full, m7, gb_all, gbt_all):
    return pl.pallas_call(
        _attn_kernel,
        grid=(B,),
        in_specs=[
            pl.BlockSpec((1, N_HASHES, 8 * H, GP // 2), lambda i: (i, 0, 0, 0)),
            pl.BlockSpec((1, N_HASHES, 8 * H, GP), lambda i: (i, 0, 0, 0)),
            pl.BlockSpec((KSIZE, GP // 2), lambda i: (0, 0)),
            pl.BlockSpec((KSIZE, GP), lambda i: (0, 0)),
            pl.BlockSpec((8, 32 * H), lambda i: (0, 0)),
            pl.BlockSpec((7, 2 * GP, N_HASHES * H), lambda i: (0, 0, 0)),
            pl.BlockSpec((7, N_HASHES * H, 2 * GP), lambda i: (0, 0, 0)),
        ],
        out_specs=[
            pl.BlockSpec((1, 32 * H, 2 * GP), lambda i: (i, 0, 0)),
            pl.BlockSpec((1, H, N_HASHES), lambda i: (i, 0, 0)),
        ],
        out_shape=[
            jax.ShapeDtypeStruct((B, 32 * H, 2 * GP), jnp.float32),
            jax.ShapeDtypeStruct((B, H, N_HASHES), jnp.float32),
        ],
    )(ws_r, vs_r, we_full, ve_full, m7, gb_all, gbt_all)


def _combine_kernel(y_ref, bsn_ref, pos_ref, out_ref):
    lane56 = jax.lax.broadcasted_iota(jnp.int32, (H, H), 1)
    for lb in range(BBS):
        ret_u = []
        bs_u = []
        for h in range(N_HASHES):
            pos_i = pos_ref[lb][:, h:h + 1]
            pt = (pos_i == lane56).astype(jnp.float32)
            ret_u.append(jnp.dot(pt, y_ref[lb][h * H:(h + 1) * H, :],
                                 preferred_element_type=jnp.float32))
            bs_u.append(jnp.dot(pt, bsn_ref[lb][:, h:h + 1],
                                preferred_element_type=jnp.float32))
        bsc = jnp.concatenate(bs_u, axis=1)
        m = jnp.max(bsc, axis=1, keepdims=True)
        e = jnp.exp(bsc - m)
        probs = e / jnp.sum(e, axis=1, keepdims=True)
        out = jnp.zeros((H, 2 * OUT), jnp.float32)
        for h in range(N_HASHES):
            out += probs[:, h:h + 1] * ret_u[h]
        out_ref[lb] = out + jnp.roll(out, -1, axis=1)


def _combine(y, bsn, pos):
    return pl.pallas_call(
        _combine_kernel,
        grid=(B // BBS,),
        in_specs=[
            pl.BlockSpec((BBS, N_HASHES * H, 2 * OUT), lambda i: (i, 0, 0)),
            pl.BlockSpec((BBS, H, N_HASHES), lambda i: (i, 0, 0)),
            pl.BlockSpec((BBS, H, N_HASHES), lambda i: (i, 0, 0)),
        ],
        out_specs=pl.BlockSpec((BBS, H, 2 * OUT), lambda i: (i, 0, 0)),
        out_shape=jax.ShapeDtypeStruct((B, H, 2 * OUT), jnp.float32),
    )(y, bsn, pos)


def kernel(x, conv_w, bn_gamma, bn_beta, relative):
    qkv_all, mean, var = _qkv_bn(x, conv_w)
    scale = bn_gamma / jnp.sqrt(var + 1e-5)
    shift = bn_beta - mean * scale

    qkv2 = jnp.transpose(qkv_all.reshape(QKV, B, H), (1, 0, 2)).reshape(
        B, H, QKV)
    scale_map = jnp.broadcast_to(scale[:, None], (QKV, H)).reshape(H, QKV)
    shift_map = jnp.broadcast_to(shift[:, None], (QKV, H)).reshape(H, QKV)
    rotf = jax.random.normal(jax.random.key(42),
                             (WM, N_HASHES, NB // 2),
                             dtype=x.dtype).reshape(WM, 2 * N_HASHES)

    sorted_w, sorted_v, pos = _sort_gather(qkv2, scale_map, shift_map, rotf)
    ws_r = sorted_w.reshape(B, N_HASHES, 8 * H, GP // 2)
    vs_r = sorted_v.reshape(B, N_HASHES, 8 * H, GP)
    we_full = relative[:, :GP // 2]
    ve_full = relative[:, GP // 2:]

    tl = jnp.arange(32 * H, dtype=jnp.int32)
    m7 = (tl[None, :] % 7 == jnp.arange(8, dtype=jnp.int32)[:, None]
          ).astype(jnp.float32)
    cl = jnp.arange(2 * GP, dtype=jnp.int32)
    rl = jnp.arange(N_HASHES * H, dtype=jnp.int32)
    bl = jnp.arange(7, dtype=jnp.int32)
    rmap = (96 * bl[:, None] + cl[None, :]) % 224
    gb_all = (rmap[:, :, None] == rl[None, None, :]).astype(jnp.float32)
    gbt_all = jnp.transpose(gb_all, (0, 2, 1))

    rbn, bsn = _attention(ws_r, vs_r, we_full, ve_full,
                          m7, gb_all, gbt_all)
    y = rbn.reshape(B, N_HASHES * H, 2 * OUT)
    s_out = _combine(y, bsn, pos)
    ret = s_out.reshape(B, H, OUT, 2)[..., 0]
    ret = ret.reshape(N, W, H, OUT)
    return jnp.transpose(ret, (0, 3, 2, 1))


# speedup vs baseline: 1.1485x; 1.1485x over previous
import jax
import jax.numpy as jnp
from jax.experimental import pallas as pl

N = 8
C_IN = 384
H = 56
W = 56
OUT = 384
N_HASHES = 4
CHUNK = 14
GROUPS = 8
KSIZE = 56
GP = OUT // GROUPS
F_GW = 0.1
F_GV1 = 0.1
F_GV2 = 1.0

B = N * W
COLS = B * H
QKV = OUT * 3 // 2
WM = OUT // 2
NB = 4
NEG = -1e30


def _qkv_kernel(w_ref, x_ref, o_ref, sum_ref, sq_ref):
    q = jnp.dot(w_ref[...], x_ref[...], preferred_element_type=jnp.float32)
    o_ref[...] = q
    s = jnp.sum(q, axis=1, keepdims=True)
    s2 = jnp.sum(q * q, axis=1, keepdims=True)

    @pl.when(pl.program_id(0) == 0)
    def _init():
        sum_ref[...] = s
        sq_ref[...] = s2

    @pl.when(pl.program_id(0) != 0)
    def _acc():
        sum_ref[...] += s
        sq_ref[...] += s2


def _qkv_bn(x, conv_w):
    xq = jnp.transpose(x, (1, 0, 3, 2)).reshape(C_IN, COLS)
    bw = 3584 if COLS % 3584 == 0 else COLS
    grid = COLS // bw
    qkv_all, ssum, ssq = pl.pallas_call(
        _qkv_kernel,
        grid=(grid,),
        in_specs=[
            pl.BlockSpec((QKV, C_IN), lambda i: (0, 0)),
            pl.BlockSpec((C_IN, bw), lambda i: (0, i)),
        ],
        out_specs=[
            pl.BlockSpec((QKV, bw), lambda i: (0, i)),
            pl.BlockSpec((QKV, 1), lambda i: (0, 0)),
            pl.BlockSpec((QKV, 1), lambda i: (0, 0)),
        ],
        out_shape=[
            jax.ShapeDtypeStruct((QKV, COLS), jnp.float32),
            jax.ShapeDtypeStruct((QKV, 1), jnp.float32),
            jax.ShapeDtypeStruct((QKV, 1), jnp.float32),
        ],
    )(conv_w, xq)
    mean = ssum[:, 0] / COLS
    var = ssq[:, 0] / COLS - mean * mean
    return qkv_all, mean, var


BBS = 4


def _sort_kernel(qkv_ref, sm_ref, sh_ref, rot_ref, sw_ref, sv_ref, pos_ref):
    bh = BBS * H
    rows = jax.lax.broadcasted_iota(jnp.int32, (bh, bh), 0)
    cols = jax.lax.broadcasted_iota(jnp.int32, (bh, bh), 1)
    lmat = ((rows >= cols) & (rows // H == cols // H)).astype(jnp.float32)
    emat = (cols == (rows // H) * H + (H - 1)).astype(jnp.float32)
    i16r = jax.lax.broadcasted_iota(jnp.int32, (4 * NB, 4 * NB), 0)
    i16c = jax.lax.broadcasted_iota(jnp.int32, (4 * NB, 4 * NB), 1)
    su16 = ((i16r // NB == i16c // NB) &
            (i16r % NB < i16c % NB)).astype(jnp.float32)
    fmat = (jax.lax.broadcasted_iota(jnp.int32, (4 * NB, N_HASHES), 0) // NB ==
            jax.lax.broadcasted_iota(jnp.int32, (4 * NB, N_HASHES), 1)
            ).astype(jnp.float32)
    lane16 = jax.lax.broadcasted_iota(jnp.int32, (bh, 4 * NB), 1)
    lane56 = jax.lax.broadcasted_iota(jnp.int32, (H, H), 1)
    sm = sm_ref[...]
    sh = sh_ref[...]
    rot = rot_ref[...]

    qns = [qkv_ref[lb] * sm + sh for lb in range(BBS)]
    wm = jnp.concatenate([qn[:, :WM] for qn in qns], axis=0)
    rotated = jnp.dot(wm, rot,
                      preferred_element_type=jnp.float32)
    oh_cols = []
    for h in range(N_HASHES):
        l0 = rotated[:, 2 * h:2 * h + 1]
        l1 = rotated[:, 2 * h + 1:2 * h + 2]
        best = l0
        bi = jnp.zeros((bh, 1), jnp.int32)
        for j, v in ((1, l1), (2, -l0), (3, -l1)):
            upd = v > best
            bi = jnp.where(upd, j, bi)
            best = jnp.maximum(best, v)
        oh_cols.append(bi)
    code16 = jnp.concatenate(
        [oh_cols[h] + NB * h for h in range(N_HASHES)], axis=1)
    onehot = jnp.zeros((bh, 4 * NB), jnp.float32)
    for h in range(N_HASHES):
        onehot += (code16[:, h:h + 1] == lane16).astype(jnp.float32)
    csum = jnp.dot(lmat, onehot,
                   preferred_element_type=jnp.float32)
    totals = jnp.dot(emat, csum,
                     preferred_element_type=jnp.float32)
    offs = jnp.dot(totals, su16,
                   preferred_element_type=jnp.float32)
    posall = jnp.dot(onehot * (offs + csum), fmat,
                     preferred_element_type=jnp.float32) - 1.0
    pos_i = posall.astype(jnp.int32)
    for lb in range(BBS):
        pos_ref[lb] = pos_i[lb * H:(lb + 1) * H, :]
        for h in range(N_HASHES):
            pc = pos_i[lb * H:(lb + 1) * H, h:h + 1]
            pt = (pc == lane56).astype(jnp.float32)
            sorted_h = jax.lax.dot_general(
                pt, qns[lb], (((0,), (0,)), ((), ())),
                preferred_element_type=jnp.float32)
            sw_ref[lb, h] = sorted_h[:, :WM]
            sv_ref[lb, h] = sorted_h[:, WM:]


def _sort_gather(qkv2, scale_map, shift_map, rotf):
    return pl.pallas_call(
        _sort_kernel,
        grid=(B // BBS,),
        in_specs=[
            pl.BlockSpec((BBS, H, QKV), lambda i: (i, 0, 0)),
            pl.BlockSpec((H, QKV), lambda i: (0, 0)),
            pl.BlockSpec((H, QKV), lambda i: (0, 0)),
            pl.BlockSpec((WM, 2 * N_HASHES), lambda i: (0, 0)),
        ],
        out_specs=[
            pl.BlockSpec((BBS, N_HASHES, H, WM), lambda i: (i, 0, 0, 0)),
            pl.BlockSpec((BBS, N_HASHES, H, OUT), lambda i: (i, 0, 0, 0)),
            pl.BlockSpec((BBS, H, N_HASHES), lambda i: (i, 0, 0)),
        ],
        out_shape=[
            jax.ShapeDtypeStruct((B, N_HASHES, H, WM), jnp.float32),
            jax.ShapeDtypeStruct((B, N_HASHES, H, OUT), jnp.float32),
            jax.ShapeDtypeStruct((B, H, N_HASHES), jnp.int32),
        ],
    )(qkv2, scale_map, shift_map, rotf)


def _attn_kernel(ws_ref, vs_ref, we_ref, ve_ref, m7_ref, gb_ref, gbt_ref,
                 out_ref, bsn_ref):
    wes = we_ref[...] * F_GW
    ves = ve_ref[...] * F_GV1
    ci = jax.lax.broadcasted_iota(jnp.int32, (H, H), 0) // CHUNK
    cj = jax.lax.broadcasted_iota(jnp.int32, (H, H), 1) // CHUNK
    allowed = cj != ((ci + 2) & 3)

    qstack = ws_ref[0].reshape(8 * N_HASHES * H, GP // 2)
    nrm = jnp.sqrt(jnp.sum(qstack * qstack, axis=1, keepdims=True))
    qn_stack = qstack / jnp.maximum(nrm, 5e-5)

    def _blk(arr, h, au):
        base = h * 8 * H + au * H
        return arr[base:base + H, :]

    sa_list = []
    for g in range(GROUPS):
        h, a = g // 2, g % 2
        for u in range(N_HASHES):
            au = a * 4 + u
            q = _blk(qstack, h, au)
            rhs = jnp.concatenate([_blk(qn_stack, h, au), wes], axis=0)
            sa_list.append(jax.lax.dot_general(
                q, rhs, (((1,), (1,)), ((), ())),
                preferred_element_type=jnp.float32))
    sa = jnp.concatenate(sa_list, axis=0).reshape(GROUPS, N_HASHES, H, 2 * H)
    ssq = jnp.sum(sa * sa, axis=0)
    den = jnp.maximum(jnp.sqrt(ssq[:, :, :H] + ssq[:, :, H:]), 5e-5)
    ss = (sa[..., :H] + sa[..., H:]) / den[None]
    ssm = jnp.where(allowed[None, None], ss, NEG)
    m = jnp.max(ssm, axis=-1, keepdims=True)
    e = jnp.exp(ssm - m)
    se = jnp.sum(e, axis=-1, keepdims=True)
    bs_all = m + jnp.log(se)
    p_all = e / se

    r96 = []
    for g in range(GROUPS):
        h, a = g // 2, g % 2
        for u in range(N_HASHES):
            au = a * 4 + u
            vcat = jnp.concatenate(
                [vs_ref[0, h][au * H:(au + 1) * H, :], ves], axis=1)
            r96.append(jnp.dot(p_all[g, u], vcat,
                               preferred_element_type=jnp.float32))
    rbig = jnp.concatenate(r96, axis=0)

    m7 = m7_ref[...]
    ssq = jnp.dot(m7, rbig * rbig,
                  preferred_element_type=jnp.float32)
    norms = jnp.zeros((1, 224), jnp.float32)
    for b in range(7):
        norms += jnp.dot(ssq[b:b + 1, :], gb_ref[b],
                         preferred_element_type=jnp.float32)
    recip = 1.0 / jnp.maximum(jnp.sqrt(norms), 5e-5)
    drows = []
    for b in range(7):
        drows.append(jnp.dot(recip, gbt_ref[b],
                             preferred_element_type=jnp.float32))
    drows.append(jnp.zeros((1, 2 * GP), jnp.float32))
    recip_block = jnp.concatenate(drows, axis=0)
    recip_full = jax.lax.dot_general(
        m7, recip_block, (((0,), (0,)), ((), ())),
        preferred_element_type=jnp.float32)
    out_ref[0] = rbig * recip_full

    sq = jnp.sum(bs_all * bs_all, axis=0)
    nb = jnp.maximum(jnp.sqrt(sq), 5e-5)
    acc = jnp.sum(bs_all, axis=0) / nb
    bsn_ref[0] = jnp.concatenate([acc[u] for u in range(N_HASHES)], axis=1)


def _attention(ws_r, vs_r, we_full, ve_full, m7, gb_all, gbt_all):
    return pl.pallas_call(
        _attn_kernel,
        grid=(B,),
        in_specs=[
            pl.BlockSpec((1, N_HASHES, 8 * H, GP // 2), lambda i: (i, 0, 0, 0)),
            pl.BlockSpec((1, N_HASHES, 8 * H, GP), lambda i: (i, 0, 0, 0)),
            pl.BlockSpec((KSIZE, GP // 2), lambda i: (0, 0)),
            pl.BlockSpec((KSIZE, GP), lambda i: (0, 0)),
            pl.BlockSpec((8, 32 * H), lambda i: (0, 0)),
            pl.BlockSpec((7, 2 * GP, N_HASHES * H), lambda i: (0, 0, 0)),
            pl.BlockSpec((7, N_HASHES * H, 2 * GP), lambda i: (0, 0, 0)),
        ],
        out_specs=[
            pl.BlockSpec((1, 32 * H, 2 * GP), lambda i: (i, 0, 0)),
            pl.BlockSpec((1, H, N_HASHES), lambda i: (i, 0, 0)),
        ],
        out_shape=[
            jax.ShapeDtypeStruct((B, 32 * H, 2 * GP), jnp.float32),
            jax.ShapeDtypeStruct((B, H, N_HASHES), jnp.float32),
        ],
    )(ws_r, vs_r, we_full, ve_full, m7, gb_all, gbt_all)


def _combine_kernel(y_ref, bsn_ref, pos_ref, out_ref):
    lane56 = jax.lax.broadcasted_iota(jnp.int32, (H, H), 1)
    for lb in range(BBS):
        ret_u = []
        bs_u = []
        for h in range(N_HASHES):
            pos_i = pos_ref[lb][:, h:h + 1]
            pt = (pos_i == lane56).astype(jnp.float32)
            ret_u.append(jnp.dot(pt, y_ref[lb][h * H:(h + 1) * H, :],
                                 preferred_element_type=jnp.float32))
            bs_u.append(jnp.dot(pt, bsn_ref[lb][:, h:h + 1],
                                preferred_element_type=jnp.float32))
        bsc = jnp.concatenate(bs_u, axis=1)
        m = jnp.max(bsc, axis=1, keepdims=True)
        e = jnp.exp(bsc - m)
        probs = e / jnp.sum(e, axis=1, keepdims=True)
        out = jnp.zeros((H, 2 * OUT), jnp.float32)
        for h in range(N_HASHES):
            out += probs[:, h:h + 1] * ret_u[h]
        out_ref[lb] = out + jnp.roll(out, -1, axis=1)


def _combine(y, bsn, pos):
    return pl.pallas_call(
        _combine_kernel,
        grid=(B // BBS,),
        in_specs=[
            pl.BlockSpec((BBS, N_HASHES * H, 2 * OUT), lambda i: (i, 0, 0)),
            pl.BlockSpec((BBS, H, N_HASHES), lambda i: (i, 0, 0)),
            pl.BlockSpec((BBS, H, N_HASHES), lambda i: (i, 0, 0)),
        ],
        out_specs=pl.BlockSpec((BBS, H, 2 * OUT), lambda i: (i, 0, 0)),
        out_shape=jax.ShapeDtypeStruct((B, H, 2 * OUT), jnp.float32),
    )(y, bsn, pos)


def kernel(x, conv_w, bn_gamma, bn_beta, relative):
    qkv_all, mean, var = _qkv_bn(x, conv_w)
    scale = bn_gamma / jnp.sqrt(var + 1e-5)
    shift = bn_beta - mean * scale

    qkv2 = jnp.transpose(qkv_all.reshape(QKV, B, H), (1, 0, 2)).reshape(
        B, H, QKV)
    scale_map = jnp.broadcast_to(scale[:, None], (QKV, H)).reshape(H, QKV)
    shift_map = jnp.broadcast_to(shift[:, None], (QKV, H)).reshape(H, QKV)
    rotf = jax.random.normal(jax.random.key(42),
                             (WM, N_HASHES, NB // 2),
                             dtype=x.dtype).reshape(WM, 2 * N_HASHES)

    sorted_w, sorted_v, pos = _sort_gather(qkv2, scale_map, shift_map, rotf)
    ws_r = sorted_w.reshape(B, N_HASHES, 8 * H, GP // 2)
    vs_r = sorted_v.reshape(B, N_HASHES, 8 * H, GP)
    we_full = relative[:, :GP // 2]
    ve_full = relative[:, GP // 2:]

    tl = jnp.arange(32 * H, dtype=jnp.int32)
    m7 = (tl[None, :] % 7 == jnp.arange(8, dtype=jnp.int32)[:, None]
          ).astype(jnp.float32)
    cl = jnp.arange(2 * GP, dtype=jnp.int32)
    rl = jnp.arange(N_HASHES * H, dtype=jnp.int32)
    bl = jnp.arange(7, dtype=jnp.int32)
    rmap = (96 * bl[:, None] + cl[None, :]) % 224
    gb_all = (rmap[:, :, None] == rl[None, None, :]).astype(jnp.float32)
    gbt_all = jnp.transpose(gb_all, (0, 2, 1))

    rbn, bsn = _attention(ws_r, vs_r, we_full, ve_full,
                          m7, gb_all, gbt_all)
    y = rbn.reshape(B, N_HASHES * H, 2 * OUT)
    s_out = _combine(y, bsn, pos)
    ret = s_out.reshape(B, H, OUT, 2)[..., 0]
    ret = ret.reshape(N, W, H, OUT)
    return jnp.transpose(ret, (0, 3, 2, 1))
